# baseline (speedup 1.0000x reference)
import functools

import jax
import jax.numpy as jnp
from jax import lax
from jax.experimental import pallas as pl
from jax.experimental.pallas import tpu as pltpu
from jax.experimental.pallas import tpu_sc as plsc

_B, _S, _D = 16, 4096, 512
_NT = 32
_NO = 32
_NTH = 16
_DD = 128
_ROW = 2 * _D + _DD
_BINS = (1, 2, 3, 4, 5, 8, 16, 32, 64)

_mesh = plsc.VectorSubcoreMesh(core_axis_name="c", subcore_axis_name="s")


@functools.partial(
    pl.kernel,
    mesh=_mesh,
    compiler_params=pltpu.CompilerParams(needs_layout_passes=False),
    out_type=jax.ShapeDtypeStruct((_B * _NT * _NO, _ROW), jnp.float32),
    scratch_types=[
        pltpu.VMEM((2 * _S,), jnp.int32),
        pltpu.VMEM((_NTH,), jnp.int32),
        pltpu.VMEM((_NO,), jnp.int32),
        pltpu.VMEM((_NTH,), jnp.int32),
        pltpu.VMEM((_NO,), jnp.int32),
        pltpu.VMEM((2 * _NTH,), jnp.int32),
        pltpu.VMEM((_NO,), jnp.int32),
        pltpu.VMEM((_NTH, _D), jnp.float32),
        pltpu.VMEM((_NO, _D), jnp.float32),
        pltpu.VMEM((_NO, _DD), jnp.float32),
        pltpu.SemaphoreType.DMA,
        pltpu.SemaphoreType.DMA,
    ],
)
def _pair_rep_sc(spans2d, sidx_hbm, ti_hbm, oi_hbm, dist_hbm, out_hbm,
                 sidx_v, tiv, oiv, tidx, oidx, tse, bidx, t_buf, o_buf,
                 emb_buf, sem_g, sem_w):
    wid = lax.axis_index("s") * 2 + lax.axis_index("c")
    b = wid // 2
    i_lo = (wid % 2) * _NTH

    pltpu.sync_copy(sidx_hbm, sidx_v)
    pltpu.sync_copy(ti_hbm.at[pl.ds(b * _NT + i_lo, _NTH)], tiv)
    pltpu.sync_copy(oi_hbm.at[pl.ds(b * _NO, _NO)], oiv)

    ti = tiv[...]
    oi0 = oiv[pl.ds(0, 16)]
    oi1 = oiv[pl.ds(16, 16)]

    base = b * _S
    tidx[...] = ti + base
    oidx[pl.ds(0, 16)] = oi0 + base
    oidx[pl.ds(16, 16)] = oi1 + base

    cp_t = pltpu.async_copy(spans2d.at[tidx], t_buf, sem_g)
    cp_o = pltpu.async_copy(spans2d.at[oidx], o_buf, sem_g)

    t_start = plsc.load_gather(sidx_v, [2 * ti])
    t_end = plsc.load_gather(sidx_v, [2 * ti + 1])
    o_start0 = plsc.load_gather(sidx_v, [2 * oi0])
    o_start1 = plsc.load_gather(sidx_v, [2 * oi1])
    o_end0 = plsc.load_gather(sidx_v, [2 * oi0 + 1])
    o_end1 = plsc.load_gather(sidx_v, [2 * oi1 + 1])

    tse[pl.ds(0, 16)] = t_start
    tse[pl.ds(16, 16)] = t_end

    cp_t.wait()
    cp_o.wait()

    def task(i, carry):
        fi = jnp.full((16,), i, jnp.int32)
        a_s = plsc.load_gather(tse, [fi])
        b_s = plsc.load_gather(tse, [fi + 16])
        buckets = []
        for o_start, o_end in ((o_start0, o_end0), (o_start1, o_end1)):
            md = jnp.minimum(jnp.abs(b_s - o_start), jnp.abs(a_s - o_end))
            bk = jnp.zeros((16,), jnp.int32)
            for t in _BINS:
                bk = bk + (md >= t).astype(jnp.int32)
            buckets.append(bk)
        bidx[pl.ds(0, 16)] = buckets[0]
        bidx[pl.ds(16, 16)] = buckets[1]
        cp_e = pltpu.async_copy(dist_hbm.at[bidx], emb_buf, sem_g)

        row0 = b * (_NT * _NO) + (i_lo + i) * _NO
        waits = []
        for j in range(_NO):
            waits.append(pltpu.async_copy(
                t_buf.at[pl.ds(i, 1)],
                out_hbm.at[pl.ds(row0 + j, 1), pl.ds(0, _D)],
                sem_w))
        waits.append(pltpu.async_copy(
            o_buf, out_hbm.at[pl.ds(row0, _NO), pl.ds(_D, _D)], sem_w))
        cp_e.wait()
        waits.append(pltpu.async_copy(
            emb_buf, out_hbm.at[pl.ds(row0, _NO), pl.ds(2 * _D, _DD)], sem_w))
        for w in waits:
            w.wait()
        return carry

    lax.fori_loop(0, _NTH, task, 0)


def kernel(spans, span_indices, target_indices, opinion_indices, dist_table):
    spans2d = spans.reshape(_B * _S, _D)
    ti = target_indices.reshape(-1).astype(jnp.int32)
    oi = opinion_indices.reshape(-1).astype(jnp.int32)
    sidx = span_indices.reshape(-1).astype(jnp.int32)
    out = _pair_rep_sc(spans2d, sidx, ti, oi, dist_table)
    return out.reshape(_B, _NT * _NO, _ROW)
